# baseline (speedup 1.0000x reference)
import functools

import jax
import jax.numpy as jnp
from jax import lax
from jax.experimental import pallas as pl
from jax.experimental.pallas import tpu as pltpu
from jax.experimental.pallas import tpu_sc as plsc

VOCAB = 1000000
EMBED_DIM = 64
BATCH = 4096
SEQ = 50

NC = 2
NS = 16
NW = NC * NS

TOTAL = BATCH * SEQ
PER_W = TOTAL // NW
GATHER = 128
ROWS_PER_CHUNK = 640
G_PER_CHUNK = ROWS_PER_CHUNK // GATHER
CHUNKS = PER_W // ROWS_PER_CHUNK
IDX_ROWS = PER_W // GATHER


def _sc_gather(idx_hbm, table_hbm, out_hbm, idx_v, rows_v, gsem):
    wid = lax.axis_index("s") * NC + lax.axis_index("c")
    pltpu.sync_copy(idx_hbm.at[wid], idx_v)
    base = wid * PER_W

    @pl.loop(0, CHUNKS)
    def _chunk(c):
        copies = []
        for j in range(G_PER_CHUNK):
            copies.append(
                pltpu.async_copy(
                    table_hbm.at[idx_v.at[c * G_PER_CHUNK + j]],
                    rows_v.at[pl.ds(j * GATHER, GATHER)],
                    gsem,
                )
            )
        for cp in copies:
            cp.wait()
        pltpu.sync_copy(
            rows_v, out_hbm.at[pl.ds(base + c * ROWS_PER_CHUNK, ROWS_PER_CHUNK)]
        )


@jax.jit
def _embed_lookup(q1, embed_weight):
    idx = q1.reshape(NW, IDX_ROWS, GATHER).astype(jnp.int32)
    run = pl.kernel(
        _sc_gather,
        out_type=jax.ShapeDtypeStruct((TOTAL, EMBED_DIM), jnp.float32),
        mesh=plsc.VectorSubcoreMesh(core_axis_name="c", subcore_axis_name="s"),
        scratch_types=[
            pltpu.VMEM((IDX_ROWS, GATHER), jnp.int32),
            pltpu.VMEM((ROWS_PER_CHUNK, EMBED_DIM), jnp.float32),
            pltpu.SemaphoreType.DMA,
        ],
        compiler_params=pltpu.CompilerParams(use_tc_tiling_on_sc=False),
    )
    out = run(idx, embed_weight)
    return out.reshape(1, TOTAL, EMBED_DIM)


def kernel(q1, embed_weight):
    return _embed_lookup(q1, embed_weight)


# speedup vs baseline: 1.0069x; 1.0069x over previous
import functools

import jax
import jax.numpy as jnp
from jax import lax
from jax.experimental import pallas as pl
from jax.experimental.pallas import tpu as pltpu
from jax.experimental.pallas import tpu_sc as plsc

VOCAB = 1000000
EMBED_DIM = 64
BATCH = 4096
SEQ = 50

NC = 2
NS = 16
NW = NC * NS

TOTAL = BATCH * SEQ
PER_W = TOTAL // NW
GATHER = 128
ROWS_PER_CHUNK = 640
G_PER_CHUNK = ROWS_PER_CHUNK // GATHER
CHUNKS = PER_W // ROWS_PER_CHUNK
IDX_ROWS = PER_W // GATHER


def _sc_gather(idx_hbm, table_hbm, out_hbm, idx_v, rows_v,
               gsem0, gsem1, wsem0, wsem1):
    wid = lax.axis_index("s") * NC + lax.axis_index("c")
    gsems = (gsem0, gsem1)
    wsems = (wsem0, wsem1)
    pltpu.sync_copy(idx_hbm.at[wid], idx_v)
    base = wid * PER_W

    def issue_gathers(c, b):
        for j in range(G_PER_CHUNK):
            pltpu.async_copy(
                table_hbm.at[idx_v.at[c * G_PER_CHUNK + j]],
                rows_v.at[b, pl.ds(j * GATHER, GATHER)],
                gsems[b],
            )

    def wait_gathers(b):
        pltpu.make_async_copy(
            table_hbm.at[pl.ds(0, ROWS_PER_CHUNK)], rows_v.at[b], gsems[b]
        ).wait()

    def out_slice(c):
        return out_hbm.at[pl.ds(base + c * ROWS_PER_CHUNK, ROWS_PER_CHUNK)]

    issue_gathers(0, 0)

    @pl.loop(0, CHUNKS, step=2)
    def _chunk(c):
        for b in range(2):
            cc = c + b
            nxt = 1 - b
            @pl.when(cc + 1 < CHUNKS)
            def _():
                @pl.when(cc >= 1)
                def _():
                    pltpu.make_async_copy(
                        rows_v.at[nxt], out_slice(cc - 1), wsems[nxt]
                    ).wait()
                issue_gathers(cc + 1, nxt)

            wait_gathers(b)
            pltpu.async_copy(rows_v.at[b], out_slice(cc), wsems[b])

    for b in range(2):
        pltpu.make_async_copy(
            rows_v.at[b], out_slice(CHUNKS - 2 + b), wsems[b]
        ).wait()


@jax.jit
def _embed_lookup(q1, embed_weight):
    idx = q1.reshape(NW, IDX_ROWS, GATHER).astype(jnp.int32)
    run = pl.kernel(
        _sc_gather,
        out_type=jax.ShapeDtypeStruct((TOTAL, EMBED_DIM), jnp.float32),
        mesh=plsc.VectorSubcoreMesh(core_axis_name="c", subcore_axis_name="s"),
        scratch_types=[
            pltpu.VMEM((IDX_ROWS, GATHER), jnp.int32),
            pltpu.VMEM((2, ROWS_PER_CHUNK, EMBED_DIM), jnp.float32),
            pltpu.SemaphoreType.DMA,
            pltpu.SemaphoreType.DMA,
            pltpu.SemaphoreType.DMA,
            pltpu.SemaphoreType.DMA,
        ],
        compiler_params=pltpu.CompilerParams(use_tc_tiling_on_sc=False),
    )
    out = run(idx, embed_weight)
    return out.reshape(1, TOTAL, EMBED_DIM)


def kernel(q1, embed_weight):
    return _embed_lookup(q1, embed_weight)


# speedup vs baseline: 1.0084x; 1.0015x over previous
import functools

import jax
import jax.numpy as jnp
from jax import lax
from jax.experimental import pallas as pl
from jax.experimental.pallas import tpu as pltpu
from jax.experimental.pallas import tpu_sc as plsc

VOCAB = 1000000
EMBED_DIM = 64
BATCH = 4096
SEQ = 50

NC = 2
NS = 16
NW = NC * NS

TOTAL = BATCH * SEQ
PER_W = TOTAL // NW
GATHER = 128
ROWS_PER_CHUNK = 640
G_PER_CHUNK = ROWS_PER_CHUNK // GATHER
CHUNKS = PER_W // ROWS_PER_CHUNK
IDX_ROWS = PER_W // GATHER


def _sc_gather(idx_hbm, table_hbm, out_hbm, idx_v, rows_v,
               gsem0, gsem1, wsem0, wsem1):
    wid = lax.axis_index("s") * NC + lax.axis_index("c")
    gsems = (gsem0, gsem1)
    wsems = (wsem0, wsem1)
    base = wid * PER_W
    pltpu.sync_copy(idx_hbm.at[pl.ds(wid * IDX_ROWS, IDX_ROWS)], idx_v)

    def issue_gathers(c, b):
        for j in range(G_PER_CHUNK):
            pltpu.async_copy(
                table_hbm.at[idx_v.at[c * G_PER_CHUNK + j]],
                rows_v.at[b, pl.ds(j * GATHER, GATHER)],
                gsems[b],
            )

    def wait_gathers(b):
        pltpu.make_async_copy(
            table_hbm.at[pl.ds(0, ROWS_PER_CHUNK)], rows_v.at[b], gsems[b]
        ).wait()

    def out_slice(c):
        return out_hbm.at[pl.ds(base + c * ROWS_PER_CHUNK, ROWS_PER_CHUNK)]

    issue_gathers(0, 0)

    @pl.loop(0, CHUNKS, step=2)
    def _chunk(c):
        for b in range(2):
            cc = c + b
            nxt = 1 - b
            @pl.when(cc + 1 < CHUNKS)
            def _():
                @pl.when(cc >= 1)
                def _():
                    pltpu.make_async_copy(
                        rows_v.at[nxt], out_slice(cc - 1), wsems[nxt]
                    ).wait()
                issue_gathers(cc + 1, nxt)

            wait_gathers(b)
            pltpu.async_copy(rows_v.at[b], out_slice(cc), wsems[b])

    for b in range(2):
        pltpu.make_async_copy(
            rows_v.at[b], out_slice(CHUNKS - 2 + b), wsems[b]
        ).wait()


@jax.jit
def _embed_lookup(q1, embed_weight):
    idx = q1.reshape(NW * IDX_ROWS, GATHER).astype(jnp.int32)
    run = pl.kernel(
        _sc_gather,
        out_type=jax.ShapeDtypeStruct((TOTAL, EMBED_DIM), jnp.float32),
        mesh=plsc.VectorSubcoreMesh(core_axis_name="c", subcore_axis_name="s"),
        scratch_types=[
            pltpu.VMEM((IDX_ROWS, GATHER), jnp.int32),
            pltpu.VMEM((2, ROWS_PER_CHUNK, EMBED_DIM), jnp.float32),
            pltpu.SemaphoreType.DMA,
            pltpu.SemaphoreType.DMA,
            pltpu.SemaphoreType.DMA,
            pltpu.SemaphoreType.DMA,
        ],
        compiler_params=pltpu.CompilerParams(use_tc_tiling_on_sc=False),
    )
    out = run(idx, embed_weight)
    return out.reshape(1, TOTAL, EMBED_DIM)


def kernel(q1, embed_weight):
    return _embed_lookup(q1, embed_weight)
